# Initial kernel scaffold; baseline (speedup 1.0000x reference)
import jax
import jax.numpy as jnp
from jax.experimental import pallas as pl


def kernel(template_feature, search_feature, template_xyz, search_xyz, template_bc, search_bc, W1, g1, be1, W2, g2, be2, W3, g3, be3, W4, g4, be4, W5, b5):
    raise NotImplementedError("write your pallas kernel here")


# speedup vs baseline: 8.2938x; 8.2938x over previous
import functools

import numpy as np
import jax
import jax.numpy as jnp
from jax import lax
from jax.experimental import pallas as pl
from jax.experimental.pallas import tpu as pltpu
from jax.experimental.pallas import tpu_sc as plsc

B, M, N = 8, 512, 1024
K = 8
F_CH = 256
H = 256
_INV = float(1.0 / np.sqrt(1.0 + 1e-5))

NW = 32
PW = (B * N) // NW
CH = 16
NCH = PW // CH
WPB = N // PW


def _dg(x, w, dx, dw):
    return lax.dot_general(x, w, (((dx,), (dw,)), ((), ())),
                           precision=lax.Precision.HIGHEST,
                           preferred_element_type=jnp.float32)


def _tc1_body(feat_ref, xyz_ref, tbc_ref, sbct_ref,
              w1a_ref, w1b_ref, w1c_ref, g1_ref, be1_ref,
              w2_ref, g2_ref, be2_ref, w3_ref, g3_ref, be3_ref,
              p_ref, idx_ref):
    b = pl.program_id(0)
    feat = feat_ref[0]
    xyz = xyz_ref[0]
    tbc = tbc_ref[0]
    sbct = sbct_ref[0]

    h = (_dg(xyz, w1a_ref[...], 1, 1) + _dg(tbc, w1b_ref[...], 1, 1)
         + _dg(feat, w1c_ref[...], 0, 1))
    h = jnp.maximum(h * (g1_ref[...] * _INV) + be1_ref[...], 0.0)
    h = _dg(h, w2_ref[...], 1, 1)
    h = jnp.maximum(h * (g2_ref[...] * _INV) + be2_ref[...], 0.0)
    h = _dg(h, w3_ref[...], 1, 1)
    h = jnp.maximum(h * (g3_ref[...] * _INV) + be3_ref[...], 0.0)
    p_ref[0] = h

    s = jnp.zeros((M, N), jnp.float32)
    for c in range(9):
        dc = tbc[:, c:c + 1] - sbct[c:c + 1, :]
        s = s + dc * dc
    s = jnp.sqrt(jnp.maximum(s, 1e-12))
    iota = lax.broadcasted_iota(jnp.int32, (M, N), 0)
    off = b * M
    for k in range(K):
        mn = jnp.min(s, axis=0, keepdims=True)
        am = jnp.min(jnp.where(s == mn, iota, M), axis=0, keepdims=True)
        idx_ref[0, pl.ds(k, 1), :] = am + off
        s = jnp.where(iota == am, jnp.float32(3e38), s)


def _tc2_body(mid_ref, w4_ref, g4_ref, be4_ref, w5_ref, b5_ref, out_ref):
    mid = mid_ref[0]
    h = _dg(w4_ref[...], mid, 1, 1)
    h = jnp.maximum(h * (g4_ref[...] * _INV) + be4_ref[...], 0.0)
    out_ref[0] = _dg(w5_ref[...], h, 1, 0) + b5_ref[...]


def _sc_gather_max_body(tbl_ref, idx_ref, out_ref, idxv, buf, outv, sem):
    wid = lax.axis_index("s") * 2 + lax.axis_index("c")
    b = wid // WPB
    n0 = pl.multiple_of((wid % WPB) * PW, PW)
    pltpu.sync_copy(idx_ref.at[b, :, pl.ds(n0, PW)], idxv)

    def chunk(cc, carry):
        cps = []
        for k in range(K):
            vidx = idxv[k, pl.ds(cc * CH, CH)]
            cps.append(pltpu.async_copy(
                tbl_ref.at[vidx], buf.at[pl.ds(k * CH, CH), :], sem))
        for cp in cps:
            cp.wait()

        def jbody(j, c2):
            for col in range(H // 16):
                sl = pl.ds(col * 16, 16)
                acc = buf[j, sl]
                for k in range(1, K):
                    acc = jnp.maximum(acc, buf[k * CH + j, sl])
                outv[j, sl] = acc
            return c2

        lax.fori_loop(0, CH, jbody, 0)
        p0 = wid * PW + cc * CH
        pltpu.sync_copy(outv, out_ref.at[pl.ds(p0, CH), :])
        return carry

    lax.fori_loop(0, NCH, chunk, 0)


@functools.cache
def _sc_gather_max():
    mesh = plsc.VectorSubcoreMesh(core_axis_name="c", subcore_axis_name="s")
    return pl.kernel(
        _sc_gather_max_body,
        mesh=mesh,
        out_type=jax.ShapeDtypeStruct((B * N, H), jnp.float32),
        scratch_types=[
            pltpu.VMEM((K, PW), jnp.int32),
            pltpu.VMEM((K * CH, H), jnp.float32),
            pltpu.VMEM((CH, H), jnp.float32),
            pltpu.SemaphoreType.DMA,
        ],
    )


def kernel(template_feature, search_feature, template_xyz, search_xyz,
           template_bc, search_bc,
           W1, g1, be1, W2, g2, be2, W3, g3, be3, W4, g4, be4, W5, b5):
    f32 = jnp.float32
    W1a, W1b, W1c = W1[:, :3], W1[:, 3:12], W1[:, 12:]
    row = lambda v: v.reshape(1, H)
    col = lambda v: v.reshape(H, 1)
    full2 = lambda a, c: pl.BlockSpec((a, c), lambda b: (0, 0))

    P, idxf = pl.pallas_call(
        _tc1_body,
        grid=(B,),
        in_specs=[
            pl.BlockSpec((1, F_CH, M), lambda b: (b, 0, 0)),
            pl.BlockSpec((1, M, 3), lambda b: (b, 0, 0)),
            pl.BlockSpec((1, M, 9), lambda b: (b, 0, 0)),
            pl.BlockSpec((1, 9, N), lambda b: (b, 0, 0)),
            full2(H, 3), full2(H, 9), full2(H, F_CH), full2(1, H), full2(1, H),
            full2(H, H), full2(1, H), full2(1, H),
            full2(H, H), full2(1, H), full2(1, H),
        ],
        out_specs=[
            pl.BlockSpec((1, M, H), lambda b: (b, 0, 0)),
            pl.BlockSpec((1, K, N), lambda b: (b, 0, 0)),
        ],
        out_shape=[
            jax.ShapeDtypeStruct((B, M, H), f32),
            jax.ShapeDtypeStruct((B, K, N), jnp.int32),
        ],
    )(template_feature, template_xyz, template_bc,
      jnp.transpose(search_bc, (0, 2, 1)),
      W1a, W1b, W1c, row(g1), row(be1),
      W2, row(g2), row(be2), W3, row(g3), row(be3))

    mid = _sc_gather_max()(P.reshape(B * M, H), idxf)

    out = pl.pallas_call(
        _tc2_body,
        grid=(B,),
        in_specs=[
            pl.BlockSpec((1, N, H), lambda b: (b, 0, 0)),
            full2(H, H), full2(H, 1), full2(H, 1), full2(H, H), full2(H, 1),
        ],
        out_specs=pl.BlockSpec((1, H, N), lambda b: (b, 0, 0)),
        out_shape=jax.ShapeDtypeStruct((B, H, N), f32),
    )(mid.reshape(B, N, H), W4, col(g4), col(be4), W5, col(b5))
    return out


# speedup vs baseline: 9.6793x; 1.1671x over previous
import functools

import numpy as np
import jax
import jax.numpy as jnp
from jax import lax
from jax.experimental import pallas as pl
from jax.experimental.pallas import tpu as pltpu
from jax.experimental.pallas import tpu_sc as plsc

B, M, N = 8, 512, 1024
K = 8
F_CH = 256
H = 256
_INV = float(1.0 / np.sqrt(1.0 + 1e-5))

NW = 32
PW = (B * N) // NW
CH = 16
NCH = PW // CH
WPB = N // PW


def _dg(x, w, dx, dw):
    return lax.dot_general(x, w, (((dx,), (dw,)), ((), ())),
                           precision=lax.Precision.HIGHEST,
                           preferred_element_type=jnp.float32)


def _tc1_body(feat_ref, xyz_ref, tbc_ref, sbct_ref,
              w1a_ref, w1b_ref, w1c_ref, g1_ref, be1_ref,
              w2_ref, g2_ref, be2_ref, w3_ref, g3_ref, be3_ref,
              p_ref, idx_ref):
    b = pl.program_id(0)
    feat = feat_ref[0]
    xyz = xyz_ref[0]
    tbc = tbc_ref[0]
    sbct = sbct_ref[0]

    h = (_dg(xyz, w1a_ref[...], 1, 1) + _dg(tbc, w1b_ref[...], 1, 1)
         + _dg(feat, w1c_ref[...], 0, 1))
    h = jnp.maximum(h * (g1_ref[...] * _INV) + be1_ref[...], 0.0)
    h = _dg(h, w2_ref[...], 1, 1)
    h = jnp.maximum(h * (g2_ref[...] * _INV) + be2_ref[...], 0.0)
    h = _dg(h, w3_ref[...], 1, 1)
    h = jnp.maximum(h * (g3_ref[...] * _INV) + be3_ref[...], 0.0)
    p_ref[0] = h

    s = jnp.zeros((M, N), jnp.float32)
    for c in range(9):
        dc = tbc[:, c:c + 1] - sbct[c:c + 1, :]
        s = s + dc * dc
    s = jnp.sqrt(jnp.maximum(s, 1e-12))
    iota = lax.broadcasted_iota(jnp.int32, (M, N), 0)
    off = b * M
    for k in range(K):
        mn = jnp.min(s, axis=0, keepdims=True)
        am = jnp.min(jnp.where(s == mn, iota, M), axis=0, keepdims=True)
        idx_ref[0, pl.ds(k, 1), :] = am + off
        s = jnp.where(iota == am, jnp.float32(3e38), s)


def _tc2_body(mid_ref, w4_ref, g4_ref, be4_ref, w5_ref, b5_ref, out_ref):
    mid = mid_ref[0]
    h = _dg(w4_ref[...], mid, 1, 1)
    h = jnp.maximum(h * (g4_ref[...] * _INV) + be4_ref[...], 0.0)
    out_ref[0] = _dg(w5_ref[...], h, 1, 0) + b5_ref[...]


def _sc_gather_max_body(tbl_ref, idx_ref, out_ref, idxv, buf, outv,
                        g0, g1, o0, o1):
    wid = lax.axis_index("s") * 2 + lax.axis_index("c")
    b = wid // WPB
    n0 = pl.multiple_of((wid % WPB) * PW, PW)
    pltpu.sync_copy(idx_ref.at[b, :, pl.ds(n0, PW)], idxv)

    def gcopy(cc, sbase, sem, k):
        vidx = idxv[k, pl.ds(cc * CH, CH)]
        return pltpu.make_async_copy(
            tbl_ref.at[vidx], buf.at[pl.ds(sbase + k * CH, CH), :], sem)

    def fire(cc, sbase, sem):
        for k in range(K):
            gcopy(cc, sbase, sem, k).start()

    def drain(cc, sbase, sem):
        for k in range(K):
            gcopy(cc, sbase, sem, k).wait()

    def ocopy(cc, obase, osem):
        return pltpu.make_async_copy(
            outv.at[pl.ds(obase, CH), :],
            out_ref.at[pl.ds(wid * PW + cc * CH, CH), :], osem)

    def compute(cc, sbase, obase, osem):
        def jbody(j, c2):
            for col in range(H // 16):
                sl = pl.ds(col * 16, 16)
                acc = buf[sbase + j, sl]
                for k in range(1, K):
                    acc = jnp.maximum(acc, buf[sbase + k * CH + j, sl])
                outv[obase + j, sl] = acc
            return c2

        lax.fori_loop(0, CH, jbody, 0)
        ocopy(cc, obase, osem).start()

    fire(0, 0, g0)

    def pair(g, carry):
        cc0 = 2 * g
        cc1 = cc0 + 1
        fire(cc1, K * CH, g1)
        drain(cc0, 0, g0)

        @pl.when(g >= 1)
        def _():
            ocopy(cc0 - 2, 0, o0).wait()

        compute(cc0, 0, 0, o0)

        @pl.when(cc0 + 2 < NCH)
        def _():
            fire(cc0 + 2, 0, g0)

        drain(cc1, K * CH, g1)

        @pl.when(g >= 1)
        def _():
            ocopy(cc1 - 2, CH, o1).wait()

        compute(cc1, K * CH, CH, o1)
        return carry

    lax.fori_loop(0, NCH // 2, pair, 0)
    ocopy(NCH - 2, 0, o0).wait()
    ocopy(NCH - 1, CH, o1).wait()


@functools.cache
def _sc_gather_max():
    mesh = plsc.VectorSubcoreMesh(core_axis_name="c", subcore_axis_name="s")
    return pl.kernel(
        _sc_gather_max_body,
        mesh=mesh,
        out_type=jax.ShapeDtypeStruct((B * N, H), jnp.float32),
        scratch_types=[
            pltpu.VMEM((K, PW), jnp.int32),
            pltpu.VMEM((2 * K * CH, H), jnp.float32),
            pltpu.VMEM((2 * CH, H), jnp.float32),
            pltpu.SemaphoreType.DMA,
            pltpu.SemaphoreType.DMA,
            pltpu.SemaphoreType.DMA,
            pltpu.SemaphoreType.DMA,
        ],
    )


def kernel(template_feature, search_feature, template_xyz, search_xyz,
           template_bc, search_bc,
           W1, g1, be1, W2, g2, be2, W3, g3, be3, W4, g4, be4, W5, b5):
    f32 = jnp.float32
    W1a, W1b, W1c = W1[:, :3], W1[:, 3:12], W1[:, 12:]
    row = lambda v: v.reshape(1, H)
    col = lambda v: v.reshape(H, 1)
    full2 = lambda a, c: pl.BlockSpec((a, c), lambda b: (0, 0))

    P, idxf = pl.pallas_call(
        _tc1_body,
        grid=(B,),
        in_specs=[
            pl.BlockSpec((1, F_CH, M), lambda b: (b, 0, 0)),
            pl.BlockSpec((1, M, 3), lambda b: (b, 0, 0)),
            pl.BlockSpec((1, M, 9), lambda b: (b, 0, 0)),
            pl.BlockSpec((1, 9, N), lambda b: (b, 0, 0)),
            full2(H, 3), full2(H, 9), full2(H, F_CH), full2(1, H), full2(1, H),
            full2(H, H), full2(1, H), full2(1, H),
            full2(H, H), full2(1, H), full2(1, H),
        ],
        out_specs=[
            pl.BlockSpec((1, M, H), lambda b: (b, 0, 0)),
            pl.BlockSpec((1, K, N), lambda b: (b, 0, 0)),
        ],
        out_shape=[
            jax.ShapeDtypeStruct((B, M, H), f32),
            jax.ShapeDtypeStruct((B, K, N), jnp.int32),
        ],
    )(template_feature, template_xyz, template_bc,
      jnp.transpose(search_bc, (0, 2, 1)),
      W1a, W1b, W1c, row(g1), row(be1),
      W2, row(g2), row(be2), W3, row(g3), row(be3))

    mid = _sc_gather_max()(P.reshape(B * M, H), idxf)

    out = pl.pallas_call(
        _tc2_body,
        grid=(B,),
        in_specs=[
            pl.BlockSpec((1, N, H), lambda b: (b, 0, 0)),
            full2(H, H), full2(H, 1), full2(H, 1), full2(H, H), full2(H, 1),
        ],
        out_specs=pl.BlockSpec((1, H, N), lambda b: (b, 0, 0)),
        out_shape=jax.ShapeDtypeStruct((B, H, N), f32),
    )(mid.reshape(B, N, H), W4, col(g4), col(be4), W5, col(b5))
    return out


# speedup vs baseline: 9.8039x; 1.0129x over previous
import functools

import numpy as np
import jax
import jax.numpy as jnp
from jax import lax
from jax.experimental import pallas as pl
from jax.experimental.pallas import tpu as pltpu
from jax.experimental.pallas import tpu_sc as plsc

B, M, N = 8, 512, 1024
K = 8
F_CH = 256
H = 256
_INV = float(1.0 / np.sqrt(1.0 + 1e-5))

NW = 32
CH = 16
NCHAIN = 2
NB = B // NCHAIN


def _dg(x, w, dx, dw):
    return lax.dot_general(x, w, (((dx,), (dw,)), ((), ())),
                           precision=lax.Precision.HIGHEST,
                           preferred_element_type=jnp.float32)


def _tc1_body(feat_ref, xyz_ref, tbc_ref, sbct_ref,
              w1a_ref, w1b_ref, w1c_ref, g1_ref, be1_ref,
              w2_ref, g2_ref, be2_ref, w3_ref, g3_ref, be3_ref,
              p_ref, idx_ref):
    b = pl.program_id(0)
    feat = feat_ref[0]
    xyz = xyz_ref[0]
    tbc = tbc_ref[0]
    sbct = sbct_ref[0]

    h = (_dg(xyz, w1a_ref[...], 1, 1) + _dg(tbc, w1b_ref[...], 1, 1)
         + _dg(feat, w1c_ref[...], 0, 1))
    h = jnp.maximum(h * (g1_ref[...] * _INV) + be1_ref[...], 0.0)
    h = _dg(h, w2_ref[...], 1, 1)
    h = jnp.maximum(h * (g2_ref[...] * _INV) + be2_ref[...], 0.0)
    h = _dg(h, w3_ref[...], 1, 1)
    h = jnp.maximum(h * (g3_ref[...] * _INV) + be3_ref[...], 0.0)
    p_ref[0] = h

    s = jnp.zeros((M, N), jnp.float32)
    for c in range(9):
        dc = tbc[:, c:c + 1] - sbct[c:c + 1, :]
        s = s + dc * dc
    s = jnp.sqrt(jnp.maximum(s, 1e-12))
    iota = lax.broadcasted_iota(jnp.int32, (M, N), 0)
    off = b * M
    for k in range(K):
        mn = jnp.min(s, axis=0, keepdims=True)
        am = jnp.min(jnp.where(s == mn, iota, M), axis=0, keepdims=True)
        idx_ref[0, pl.ds(k, 1), :] = am + off
        s = jnp.where(iota == am, jnp.float32(3e38), s)


def _tc2_body(mid_ref, w4_ref, g4_ref, be4_ref, w5_ref, b5_ref, out_ref):
    mid = mid_ref[0]
    h = _dg(w4_ref[...], mid, 1, 1)
    h = jnp.maximum(h * (g4_ref[...] * _INV) + be4_ref[...], 0.0)
    out_ref[0] = _dg(w5_ref[...], h, 1, 0) + b5_ref[...]


def _sc_gather_max_body(nb, tbl_ref, idx_ref, out_ref, idxv, buf, outv,
                        g0, g1, o0, o1):
    pw = (nb * N) // NW
    nch = pw // CH
    wpb = N // pw
    wid = lax.axis_index("s") * 2 + lax.axis_index("c")
    b = wid // wpb
    n0 = pl.multiple_of((wid % wpb) * pw, pw)
    pltpu.sync_copy(idx_ref.at[b, :, pl.ds(n0, pw)], idxv)

    def gcopy(cc, sbase, sem, k):
        vidx = idxv[k, pl.ds(cc * CH, CH)]
        return pltpu.make_async_copy(
            tbl_ref.at[vidx], buf.at[pl.ds(sbase + k * CH, CH), :], sem)

    def fire(cc, sbase, sem):
        for k in range(K):
            gcopy(cc, sbase, sem, k).start()

    def drain(cc, sbase, sem):
        for k in range(K):
            gcopy(cc, sbase, sem, k).wait()

    def ocopy(cc, obase, osem):
        return pltpu.make_async_copy(
            outv.at[pl.ds(obase, CH), :],
            out_ref.at[pl.ds(wid * pw + cc * CH, CH), :], osem)

    def compute(cc, sbase, obase, osem):
        def jbody(j, c2):
            for col in range(H // 16):
                sl = pl.ds(col * 16, 16)
                acc = buf[sbase + j, sl]
                for k in range(1, K):
                    acc = jnp.maximum(acc, buf[sbase + k * CH + j, sl])
                outv[obase + j, sl] = acc
            return c2

        lax.fori_loop(0, CH, jbody, 0)
        ocopy(cc, obase, osem).start()

    fire(0, 0, g0)

    def pair(g, carry):
        cc0 = 2 * g
        cc1 = cc0 + 1
        fire(cc1, K * CH, g1)
        drain(cc0, 0, g0)

        @pl.when(g >= 1)
        def _():
            ocopy(cc0 - 2, 0, o0).wait()

        compute(cc0, 0, 0, o0)

        @pl.when(cc0 + 2 < nch)
        def _():
            fire(cc0 + 2, 0, g0)

        drain(cc1, K * CH, g1)

        @pl.when(g >= 1)
        def _():
            ocopy(cc1 - 2, CH, o1).wait()

        compute(cc1, K * CH, CH, o1)
        return carry

    lax.fori_loop(0, nch // 2, pair, 0)
    ocopy(nch - 2, 0, o0).wait()
    ocopy(nch - 1, CH, o1).wait()


@functools.cache
def _sc_gather_max(nb):
    pw = (nb * N) // NW
    mesh = plsc.VectorSubcoreMesh(core_axis_name="c", subcore_axis_name="s")
    return pl.kernel(
        functools.partial(_sc_gather_max_body, nb),
        mesh=mesh,
        out_type=jax.ShapeDtypeStruct((nb * N, H), jnp.float32),
        scratch_types=[
            pltpu.VMEM((K, pw), jnp.int32),
            pltpu.VMEM((2 * K * CH, H), jnp.float32),
            pltpu.VMEM((2 * CH, H), jnp.float32),
            pltpu.SemaphoreType.DMA,
            pltpu.SemaphoreType.DMA,
            pltpu.SemaphoreType.DMA,
            pltpu.SemaphoreType.DMA,
        ],
    )


def _chain(nb, tf, txyz, tbc, sbct,
           W1a, W1b, W1c, g1, be1, W2, g2, be2, W3, g3, be3,
           W4, g4, be4, W5, b5):
    f32 = jnp.float32
    full2 = lambda a, c: pl.BlockSpec((a, c), lambda b: (0, 0))

    P, idxf = pl.pallas_call(
        _tc1_body,
        grid=(nb,),
        in_specs=[
            pl.BlockSpec((1, F_CH, M), lambda b: (b, 0, 0)),
            pl.BlockSpec((1, M, 3), lambda b: (b, 0, 0)),
            pl.BlockSpec((1, M, 9), lambda b: (b, 0, 0)),
            pl.BlockSpec((1, 9, N), lambda b: (b, 0, 0)),
            full2(H, 3), full2(H, 9), full2(H, F_CH), full2(1, H), full2(1, H),
            full2(H, H), full2(1, H), full2(1, H),
            full2(H, H), full2(1, H), full2(1, H),
        ],
        out_specs=[
            pl.BlockSpec((1, M, H), lambda b: (b, 0, 0)),
            pl.BlockSpec((1, K, N), lambda b: (b, 0, 0)),
        ],
        out_shape=[
            jax.ShapeDtypeStruct((nb, M, H), f32),
            jax.ShapeDtypeStruct((nb, K, N), jnp.int32),
        ],
    )(tf, txyz, tbc, sbct,
      W1a, W1b, W1c, g1, be1, W2, g2, be2, W3, g3, be3)

    mid = _sc_gather_max(nb)(P.reshape(nb * M, H), idxf)

    return pl.pallas_call(
        _tc2_body,
        grid=(nb,),
        in_specs=[
            pl.BlockSpec((1, N, H), lambda b: (b, 0, 0)),
            full2(H, H), full2(H, 1), full2(H, 1), full2(H, H), full2(H, 1),
        ],
        out_specs=pl.BlockSpec((1, H, N), lambda b: (b, 0, 0)),
        out_shape=jax.ShapeDtypeStruct((nb, H, N), f32),
    )(mid.reshape(nb, N, H), W4, g4, be4, W5, b5)


def kernel(template_feature, search_feature, template_xyz, search_xyz,
           template_bc, search_bc,
           W1, g1, be1, W2, g2, be2, W3, g3, be3, W4, g4, be4, W5, b5):
    W1a, W1b, W1c = W1[:, :3], W1[:, 3:12], W1[:, 12:]
    row = lambda v: v.reshape(1, H)
    col = lambda v: v.reshape(H, 1)
    sbct = jnp.transpose(search_bc, (0, 2, 1))
    wargs = (W1a, W1b, W1c, row(g1), row(be1), W2, row(g2), row(be2),
             W3, row(g3), row(be3), W4, col(g4), col(be4), W5, col(b5))
    sl = lambda a, i: lax.slice_in_dim(a, i * NB, (i + 1) * NB, axis=0)
    outs = [
        _chain(NB, sl(template_feature, i), sl(template_xyz, i),
               sl(template_bc, i), sl(sbct, i), *wargs)
        for i in range(NCHAIN)
    ]
    return jnp.concatenate(outs, axis=0)


# speedup vs baseline: 10.2171x; 1.0421x over previous
import functools

import numpy as np
import jax
import jax.numpy as jnp
from jax import lax
from jax.experimental import pallas as pl
from jax.experimental.pallas import tpu as pltpu
from jax.experimental.pallas import tpu_sc as plsc

B, M, N = 8, 512, 1024
K = 8
F_CH = 256
H = 256
_INV = float(1.0 / np.sqrt(1.0 + 1e-5))

NW = 32
CH = 16
NCHAIN = 4
NB = B // NCHAIN


def _dg(x, w, dx, dw):
    return lax.dot_general(x, w, (((dx,), (dw,)), ((), ())),
                           precision=lax.Precision.HIGHEST,
                           preferred_element_type=jnp.float32)


def _tc1_body(feat_ref, xyz_ref, tbc_ref, sbct_ref,
              w1a_ref, w1b_ref, w1c_ref, g1_ref, be1_ref,
              w2_ref, g2_ref, be2_ref, w3_ref, g3_ref, be3_ref,
              p_ref, idx_ref):
    b = pl.program_id(0)
    feat = feat_ref[0]
    xyz = xyz_ref[0]
    tbc = tbc_ref[0]
    sbct = sbct_ref[0]

    h = (_dg(xyz, w1a_ref[...], 1, 1) + _dg(tbc, w1b_ref[...], 1, 1)
         + _dg(feat, w1c_ref[...], 0, 1))
    h = jnp.maximum(h * (g1_ref[...] * _INV) + be1_ref[...], 0.0)
    h = _dg(h, w2_ref[...], 1, 1)
    h = jnp.maximum(h * (g2_ref[...] * _INV) + be2_ref[...], 0.0)
    h = _dg(h, w3_ref[...], 1, 1)
    h = jnp.maximum(h * (g3_ref[...] * _INV) + be3_ref[...], 0.0)
    p_ref[0] = h

    s = jnp.zeros((M, N), jnp.float32)
    for c in range(9):
        dc = tbc[:, c:c + 1] - sbct[c:c + 1, :]
        s = s + dc * dc
    s = jnp.sqrt(jnp.maximum(s, 1e-12))
    iota = lax.broadcasted_iota(jnp.int32, (M, N), 0)
    off = b * M
    for k in range(K):
        mn = jnp.min(s, axis=0, keepdims=True)
        am = jnp.min(jnp.where(s == mn, iota, M), axis=0, keepdims=True)
        idx_ref[0, pl.ds(k, 1), :] = am + off
        s = jnp.where(iota == am, jnp.float32(3e38), s)


def _tc2_body(mid_ref, w4_ref, g4_ref, be4_ref, w5_ref, b5_ref, out_ref):
    mid = mid_ref[0]
    h = _dg(w4_ref[...], mid, 1, 1)
    h = jnp.maximum(h * (g4_ref[...] * _INV) + be4_ref[...], 0.0)
    out_ref[0] = _dg(w5_ref[...], h, 1, 0) + b5_ref[...]


def _sc_gather_max_body(nb, tbl_ref, idx_ref, out_ref, idxv, buf, outv,
                        g0, g1, o0, o1):
    pw = (nb * N) // NW
    nch = pw // CH
    wpb = N // pw
    wid = lax.axis_index("s") * 2 + lax.axis_index("c")
    b = wid // wpb
    n0 = pl.multiple_of((wid % wpb) * pw, CH)
    pltpu.sync_copy(idx_ref.at[b], idxv)

    def gcopy(cc, sbase, sem, k):
        vidx = idxv[k, pl.ds(n0 + cc * CH, CH)]
        return pltpu.make_async_copy(
            tbl_ref.at[vidx], buf.at[pl.ds(sbase + k * CH, CH), :], sem)

    def fire(cc, sbase, sem):
        for k in range(K):
            gcopy(cc, sbase, sem, k).start()

    def drain(cc, sbase, sem):
        for k in range(K):
            gcopy(cc, sbase, sem, k).wait()

    def ocopy(cc, obase, osem):
        return pltpu.make_async_copy(
            outv.at[pl.ds(obase, CH), :],
            out_ref.at[pl.ds(wid * pw + cc * CH, CH), :], osem)

    def compute(cc, sbase, obase, osem):
        def jbody(j, c2):
            for col in range(H // 16):
                sl = pl.ds(col * 16, 16)
                acc = buf[sbase + j, sl]
                for k in range(1, K):
                    acc = jnp.maximum(acc, buf[sbase + k * CH + j, sl])
                outv[obase + j, sl] = acc
            return c2

        lax.fori_loop(0, CH, jbody, 0)
        ocopy(cc, obase, osem).start()

    fire(0, 0, g0)

    def pair(g, carry):
        cc0 = 2 * g
        cc1 = cc0 + 1
        fire(cc1, K * CH, g1)
        drain(cc0, 0, g0)

        @pl.when(g >= 1)
        def _():
            ocopy(cc0 - 2, 0, o0).wait()

        compute(cc0, 0, 0, o0)

        @pl.when(cc0 + 2 < nch)
        def _():
            fire(cc0 + 2, 0, g0)

        drain(cc1, K * CH, g1)

        @pl.when(g >= 1)
        def _():
            ocopy(cc1 - 2, CH, o1).wait()

        compute(cc1, K * CH, CH, o1)
        return carry

    lax.fori_loop(0, nch // 2, pair, 0)
    ocopy(nch - 2, 0, o0).wait()
    ocopy(nch - 1, CH, o1).wait()


@functools.cache
def _sc_gather_max(nb):
    pw = (nb * N) // NW
    mesh = plsc.VectorSubcoreMesh(core_axis_name="c", subcore_axis_name="s")
    return pl.kernel(
        functools.partial(_sc_gather_max_body, nb),
        mesh=mesh,
        out_type=jax.ShapeDtypeStruct((nb * N, H), jnp.float32),
        scratch_types=[
            pltpu.VMEM((K, N), jnp.int32),
            pltpu.VMEM((2 * K * CH, H), jnp.float32),
            pltpu.VMEM((2 * CH, H), jnp.float32),
            pltpu.SemaphoreType.DMA,
            pltpu.SemaphoreType.DMA,
            pltpu.SemaphoreType.DMA,
            pltpu.SemaphoreType.DMA,
        ],
    )


def _chain(ci, nb, tf, txyz, tbc, sbct,
           W1a, W1b, W1c, g1, be1, W2, g2, be2, W3, g3, be3,
           W4, g4, be4, W5, b5):
    f32 = jnp.float32
    full2 = lambda a, c: pl.BlockSpec((a, c), lambda b: (0, 0))
    boff = lambda b: (ci * nb + b, 0, 0)

    P, idxf = pl.pallas_call(
        _tc1_body,
        grid=(nb,),
        in_specs=[
            pl.BlockSpec((1, F_CH, M), boff),
            pl.BlockSpec((1, M, 3), boff),
            pl.BlockSpec((1, M, 9), boff),
            pl.BlockSpec((1, 9, N), boff),
            full2(H, 3), full2(H, 9), full2(H, F_CH), full2(1, H), full2(1, H),
            full2(H, H), full2(1, H), full2(1, H),
            full2(H, H), full2(1, H), full2(1, H),
        ],
        out_specs=[
            pl.BlockSpec((1, M, H), lambda b: (b, 0, 0)),
            pl.BlockSpec((1, K, N), lambda b: (b, 0, 0)),
        ],
        out_shape=[
            jax.ShapeDtypeStruct((nb, M, H), f32),
            jax.ShapeDtypeStruct((nb, K, N), jnp.int32),
        ],
    )(tf, txyz, tbc, sbct,
      W1a, W1b, W1c, g1, be1, W2, g2, be2, W3, g3, be3)

    mid = _sc_gather_max(nb)(P.reshape(nb * M, H), idxf)

    return pl.pallas_call(
        _tc2_body,
        grid=(nb,),
        in_specs=[
            pl.BlockSpec((1, N, H), lambda b: (b, 0, 0)),
            full2(H, H), full2(H, 1), full2(H, 1), full2(H, H), full2(H, 1),
        ],
        out_specs=pl.BlockSpec((1, H, N), lambda b: (b, 0, 0)),
        out_shape=jax.ShapeDtypeStruct((nb, H, N), f32),
    )(mid.reshape(nb, N, H), W4, g4, be4, W5, b5)


def kernel(template_feature, search_feature, template_xyz, search_xyz,
           template_bc, search_bc,
           W1, g1, be1, W2, g2, be2, W3, g3, be3, W4, g4, be4, W5, b5):
    W1a, W1b, W1c = W1[:, :3], W1[:, 3:12], W1[:, 12:]
    row = lambda v: v.reshape(1, H)
    col = lambda v: v.reshape(H, 1)
    sbct = jnp.transpose(search_bc, (0, 2, 1))
    wargs = (W1a, W1b, W1c, row(g1), row(be1), W2, row(g2), row(be2),
             W3, row(g3), row(be3), W4, col(g4), col(be4), W5, col(b5))
    outs = [
        _chain(i, NB, template_feature, template_xyz, template_bc, sbct,
               *wargs)
        for i in range(NCHAIN)
    ]
    return jnp.concatenate(outs, axis=0)
